# baseline (speedup 1.0000x reference)
import functools

import jax
import jax.numpy as jnp
from jax import lax
from jax.experimental import pallas as pl
from jax.experimental.pallas import tpu as pltpu
from jax.experimental.pallas import tpu_sc as plsc

N = 10000
D = 128
OUT = 128
E = 320000

NC = 2
NS = 16
L = 16
NW = NC * NS
EPT = E // NW
B = 32
CH = 18
NCH = 18
NIT = NCH * CH
EPTP = NIT * B
DP = D + L
NP = 10240
RPT = NP // NS

_mesh = plsc.VectorSubcoreMesh(core_axis_name="c", subcore_axis_name="s")


@functools.partial(
    pl.kernel,
    out_type=jax.ShapeDtypeStruct((NC, NP, DP), jnp.float32),
    mesh=_mesh,
    compiler_params=pltpu.CompilerParams(use_tc_tiling_on_sc=False),
    scratch_types=[
        pltpu.VMEM((2, CH, B), jnp.int32),
        pltpu.VMEM((2, CH, B), jnp.int32),
        pltpu.VMEM((2, B, DP), jnp.float32),
        pltpu.VMEM_SHARED((NP, DP), jnp.float32),
        pltpu.SemaphoreType.DMA,
        pltpu.SemaphoreType.DMA,
    ],
)
def _sc_segment_sum(xp_hbm, src_hbm, dst_hbm, parts_hbm,
                    src_c, dst_c, rows_v, acc_sh, gsem, isem):
    cid = lax.axis_index("c")
    sid = lax.axis_index("s")
    wid = sid * NC + cid

    def _fetch_idx(ch, buf):
        pltpu.async_copy(src_hbm.at[wid, ch], src_c.at[buf], isem)
        pltpu.async_copy(dst_hbm.at[wid, ch], dst_c.at[buf], isem)

    def _wait_idx(ch, buf):
        pltpu.make_async_copy(src_hbm.at[wid, ch], src_c.at[buf], isem).wait()
        pltpu.make_async_copy(dst_hbm.at[wid, ch], dst_c.at[buf], isem).wait()

    _fetch_idx(0, 0)
    _fetch_idx(1, 1)

    zvec = jnp.zeros((L,), jnp.float32)

    def _zero_row(r, _):
        for j in range(DP // L):
            rows_v[0, r, pl.ds(j * L, L)] = zvec
        return 0

    lax.fori_loop(0, 64, _zero_row, 0)
    for k in range(RPT // 64):
        pltpu.sync_copy(rows_v.at[0, pl.ds(0, 64)],
                        acc_sh.at[pl.ds(sid * RPT + k * 64, 64)])
    plsc.subcore_barrier()

    _wait_idx(0, 0)
    pltpu.async_copy(xp_hbm.at[src_c.at[0, 0]], rows_v.at[0], gsem)

    def _pair(pair, _):
        for p in range(2):
            ch = pair * 2 + p
            cb = p

            @pl.when(ch + 1 < NCH)
            def _():
                _wait_idx(ch + 1, 1 - cb)

            for j in range(CH):
                g = ch * CH + j
                rb = j % 2
                if j + 1 < CH:
                    nidx = src_c.at[cb, j + 1]
                else:
                    nidx = src_c.at[1 - cb, 0]

                @pl.when(g + 1 < NIT)
                def _():
                    pltpu.async_copy(xp_hbm.at[nidx], rows_v.at[1 - rb], gsem)

                pltpu.make_async_copy(xp_hbm.at[src_c.at[cb, j]],
                                      rows_v.at[rb], gsem).wait()
                pltpu.sync_copy(rows_v.at[rb], acc_sh.at[dst_c.at[cb, j]],
                                add=True)

            @pl.when(ch + 2 < NCH)
            def _():
                _fetch_idx(ch + 2, cb)
        return 0

    lax.fori_loop(0, NCH // 2, _pair, 0)
    plsc.subcore_barrier()

    pltpu.sync_copy(acc_sh.at[pl.ds(sid * RPT, RPT)],
                    parts_hbm.at[cid, pl.ds(sid * RPT, RPT)])


RB = 400


def _tc_body(parts_ref, x_ref, x2_ref, wl_ref, wr_ref, bl_ref,
             out_ref, out2_ref):
    p = parts_ref[0] + parts_ref[1]
    agg = p[:, :D]
    cnt = p[:, D:D + 1]
    scale = 1.0 / jnp.maximum(cnt, 1.0)
    wl = wl_ref[...]
    wr = wr_ref[...]
    bl = bl_ref[...]
    dn = (((1,), (1,)), ((), ()))
    t = lax.dot_general(agg, wl, dn, preferred_element_type=jnp.float32)
    xr = lax.dot_general(x_ref[...], wr, dn, preferred_element_type=jnp.float32)
    out_ref[...] = t * scale + bl + xr
    t2 = lax.dot_general(x2_ref[...], wl + wr, dn,
                         preferred_element_type=jnp.float32)
    out2_ref[...] = t2 + bl


_tc_combine = pl.pallas_call(
    _tc_body,
    grid=(N // RB,),
    in_specs=[
        pl.BlockSpec((NC, RB, DP), lambda i: (0, i, 0)),
        pl.BlockSpec((RB, D), lambda i: (i, 0)),
        pl.BlockSpec((RB, D), lambda i: (i, 0)),
        pl.BlockSpec((OUT, D), lambda i: (0, 0)),
        pl.BlockSpec((OUT, D), lambda i: (0, 0)),
        pl.BlockSpec((1, OUT), lambda i: (0, 0)),
    ],
    out_specs=[
        pl.BlockSpec((RB, OUT), lambda i: (i, 0)),
        pl.BlockSpec((RB, OUT), lambda i: (i, 0)),
    ],
    out_shape=[
        jax.ShapeDtypeStruct((N, OUT), jnp.float32),
        jax.ShapeDtypeStruct((N, OUT), jnp.float32),
    ],
)


@jax.jit
def kernel(x, x_, W_l, b_l, W_r, edge_index):
    xp = jnp.concatenate([x, jnp.ones((N, L), jnp.float32)], axis=1)
    ei = edge_index.reshape(2, NW, EPT)
    pad = EPTP - EPT
    src = jnp.concatenate(
        [ei[0], jnp.zeros((NW, pad), jnp.int32)], axis=1
    ).reshape(NW, NCH, CH, B)
    dst = jnp.concatenate(
        [ei[1], jnp.full((NW, pad), NP - 1, jnp.int32)], axis=1
    ).reshape(NW, NCH, CH, B)
    parts = _sc_segment_sum(xp, src, dst)
    out, out_ = _tc_combine(parts, x, x_, W_l, W_r, b_l.reshape(1, OUT))
    return (out, out_)


# speedup vs baseline: 3.9480x; 3.9480x over previous
import functools

import jax
import jax.numpy as jnp
from jax import lax
from jax.experimental import pallas as pl
from jax.experimental.pallas import tpu as pltpu
from jax.experimental.pallas import tpu_sc as plsc

N = 10000
D = 128
OUT = 128
E = 320000

NC = 2
NS = 16
L = 16
NW = NC * NS
EPT = E // NW
B = 64
CH = 12
NCH = 13
NIT = NCH * CH
CHB = CH * B
TB = EPT - NIT * B
NP = 10240
RPT = NP // NS

_mesh = plsc.VectorSubcoreMesh(core_axis_name="c", subcore_axis_name="s")


@functools.partial(
    pl.kernel,
    out_type=[
        jax.ShapeDtypeStruct((NC, NP, D), jnp.float32),
        jax.ShapeDtypeStruct((NC, NP, L), jnp.float32),
    ],
    mesh=_mesh,
    compiler_params=pltpu.CompilerParams(use_tc_tiling_on_sc=False),
    scratch_types=[
        pltpu.VMEM((2, CHB), jnp.int32),
        pltpu.VMEM((2, CHB), jnp.int32),
        pltpu.VMEM((2, B, D), jnp.float32),
        pltpu.VMEM((B, L), jnp.float32),
        pltpu.VMEM((B, L), jnp.float32),
        pltpu.VMEM_SHARED((NP, D), jnp.float32),
        pltpu.VMEM_SHARED((NP, L), jnp.float32),
        pltpu.SemaphoreType.DMA,
        pltpu.SemaphoreType.DMA,
        pltpu.SemaphoreType.DMA,
    ],
)
def _sc_segment_sum(x_hbm, ei_hbm, agg_hbm, cnt_hbm,
                    src_c, dst_c, rows_v, ones_v, zb_v,
                    acc_sh, cac_sh, gsem, isem, csem):
    cid = lax.axis_index("c")
    sid = lax.axis_index("s")
    wid = sid * NC + cid
    base = wid * EPT

    def _fetch_idx(ch, buf):
        pltpu.async_copy(ei_hbm.at[0, pl.ds(base + ch * CHB, CHB)],
                         src_c.at[buf], isem)
        pltpu.async_copy(ei_hbm.at[1, pl.ds(base + ch * CHB, CHB)],
                         dst_c.at[buf], isem)

    def _wait_idx(ch, buf):
        pltpu.make_async_copy(ei_hbm.at[0, pl.ds(base + ch * CHB, CHB)],
                              src_c.at[buf], isem).wait()
        pltpu.make_async_copy(ei_hbm.at[1, pl.ds(base + ch * CHB, CHB)],
                              dst_c.at[buf], isem).wait()

    _fetch_idx(0, 0)
    _fetch_idx(1, 1)

    zvec = jnp.zeros((L,), jnp.float32)
    ovec = jnp.ones((L,), jnp.float32)

    def _zero_row(r, _):
        for j in range(D // L):
            rows_v[0, r, pl.ds(j * L, L)] = zvec
        ones_v[r] = ovec
        zb_v[r] = zvec
        return 0

    lax.fori_loop(0, B, _zero_row, 0)
    for k in range(RPT // B):
        pltpu.sync_copy(rows_v.at[0], acc_sh.at[pl.ds(sid * RPT + k * B, B)])
        pltpu.sync_copy(zb_v, cac_sh.at[pl.ds(sid * RPT + k * B, B)])
    plsc.subcore_barrier()

    _wait_idx(0, 0)
    pltpu.async_copy(x_hbm.at[src_c.at[0, pl.ds(0, B)]], rows_v.at[0], gsem)

    def _chunk(ch, cb):
        @pl.when(ch + 1 < NCH)
        def _():
            _wait_idx(ch + 1, 1 - cb)

        for j in range(CH):
            g = ch * CH + j
            rb = j % 2
            if j + 1 < CH:
                nidx = src_c.at[cb, pl.ds((j + 1) * B, B)]
            else:
                nidx = src_c.at[1 - cb, pl.ds(0, B)]

            @pl.when(g + 1 < NIT)
            def _():
                pltpu.async_copy(x_hbm.at[nidx], rows_v.at[1 - rb], gsem)

            didx = dst_c.at[cb, pl.ds(j * B, B)]
            pltpu.make_async_copy(x_hbm.at[src_c.at[cb, pl.ds(j * B, B)]],
                                  rows_v.at[rb], gsem).wait()
            pltpu.async_copy(ones_v, cac_sh.at[didx], csem, add=True)
            pltpu.sync_copy(rows_v.at[rb], acc_sh.at[didx], add=True)
            pltpu.make_async_copy(ones_v, cac_sh.at[didx], csem).wait()

        @pl.when(ch + 2 < NCH)
        def _():
            _fetch_idx(ch + 2, cb)

    def _pair(pair, _):
        for p in range(2):
            _chunk(pair * 2 + p, p)
        return 0

    lax.fori_loop(0, NCH // 2, _pair, 0)
    _chunk(NCH - 1, (NCH - 1) % 2)

    tsrc = ei_hbm.at[0, pl.ds(base + NIT * B, TB)]
    tdst = ei_hbm.at[1, pl.ds(base + NIT * B, TB)]
    pltpu.sync_copy(tsrc, src_c.at[0, pl.ds(0, TB)])
    pltpu.sync_copy(tdst, dst_c.at[0, pl.ds(0, TB)])
    tidx = src_c.at[0, pl.ds(0, TB)]
    tdidx = dst_c.at[0, pl.ds(0, TB)]
    pltpu.sync_copy(x_hbm.at[tidx], rows_v.at[0, pl.ds(0, TB)])
    pltpu.sync_copy(rows_v.at[0, pl.ds(0, TB)], acc_sh.at[tdidx], add=True)
    pltpu.sync_copy(ones_v.at[pl.ds(0, TB)], cac_sh.at[tdidx], add=True)
    plsc.subcore_barrier()

    pltpu.sync_copy(acc_sh.at[pl.ds(sid * RPT, RPT)],
                    agg_hbm.at[cid, pl.ds(sid * RPT, RPT)])
    pltpu.sync_copy(cac_sh.at[pl.ds(sid * RPT, RPT)],
                    cnt_hbm.at[cid, pl.ds(sid * RPT, RPT)])


RB = 400


def _tc_body(agg_ref, cnt_ref, x_ref, x2_ref, wl_ref, wr_ref, bl_ref,
             out_ref, out2_ref):
    agg = agg_ref[0] + agg_ref[1]
    c = cnt_ref[0] + cnt_ref[1]
    cnt = c[:, 0:1]
    scale = 1.0 / jnp.maximum(cnt, 1.0)
    wl = wl_ref[...]
    wr = wr_ref[...]
    bl = bl_ref[...]
    dn = (((1,), (1,)), ((), ()))
    t = lax.dot_general(agg, wl, dn, preferred_element_type=jnp.float32)
    xr = lax.dot_general(x_ref[...], wr, dn, preferred_element_type=jnp.float32)
    out_ref[...] = t * scale + bl + xr
    t2 = lax.dot_general(x2_ref[...], wl + wr, dn,
                         preferred_element_type=jnp.float32)
    out2_ref[...] = t2 + bl


_tc_combine = pl.pallas_call(
    _tc_body,
    grid=(N // RB,),
    in_specs=[
        pl.BlockSpec((NC, RB, D), lambda i: (0, i, 0)),
        pl.BlockSpec((NC, RB, L), lambda i: (0, i, 0)),
        pl.BlockSpec((RB, D), lambda i: (i, 0)),
        pl.BlockSpec((RB, D), lambda i: (i, 0)),
        pl.BlockSpec((OUT, D), lambda i: (0, 0)),
        pl.BlockSpec((OUT, D), lambda i: (0, 0)),
        pl.BlockSpec((1, OUT), lambda i: (0, 0)),
    ],
    out_specs=[
        pl.BlockSpec((RB, OUT), lambda i: (i, 0)),
        pl.BlockSpec((RB, OUT), lambda i: (i, 0)),
    ],
    out_shape=[
        jax.ShapeDtypeStruct((N, OUT), jnp.float32),
        jax.ShapeDtypeStruct((N, OUT), jnp.float32),
    ],
)


@jax.jit
def kernel(x, x_, W_l, b_l, W_r, edge_index):
    agg, cnt = _sc_segment_sum(x, edge_index)
    out, out_ = _tc_combine(agg, cnt, x, x_, W_l, W_r, b_l.reshape(1, OUT))
    return (out, out_)
